# baseline (speedup 1.0000x reference)
import jax
import jax.numpy as jnp
from jax.experimental import pallas as pl
from jax.experimental.pallas import tpu as pltpu
from jax.experimental.pallas import tpu_sc as plsc

IN_DIM = 128
OUT_DIM = 256
MEM_LEN = 100000
BETA = 2000.0

BLK = 5000
NBLK = MEM_LEN // BLK

SC_CORES = 2
SC_CHUNKS = 10
SC_ROWS = MEM_LEN // SC_CORES
SC_CHUNK_ROWS = SC_ROWS // SC_CHUNKS


def _tc_body(pos_ref, x_ref, mean_ref, std_ref, w_ref, b_ref, mem_ref,
             loss_ref, out_mem_ref, enc_ref, min_ref):
    i = pl.program_id(0)

    @pl.when(i == 0)
    def _encode():
        xv = x_ref[...]
        std = std_ref[...]
        new = jnp.where(std == 0.0, 0.0, (xv - mean_ref[...]) / std)
        logits = jnp.dot(new, w_ref[...],
                         preferred_element_type=jnp.float32) + b_ref[...]
        m = jnp.max(logits)
        lse = jnp.log(jnp.sum(jnp.exp(logits - m))) + m
        enc_ref[...] = logits - lse
        min_ref[0] = jnp.inf

    @pl.when(i < NBLK)
    def _stream():
        blk = mem_ref[...]
        out_mem_ref[...] = blk
        d = jnp.sum(jnp.abs(blk - enc_ref[...]), axis=1)
        min_ref[0] = jnp.minimum(min_ref[0], jnp.min(d))

    @pl.when(i == NBLK)
    def _fixup():
        loss = min_ref[0]
        loss_ref[...] = jnp.full((1, 1), loss, jnp.float32)
        do_update = loss <= BETA
        r = pos_ref[0] % BLK
        row_sel = jax.lax.broadcasted_iota(jnp.int32, (BLK, 1), 0) == r
        sel = jnp.logical_and(do_update, row_sel)
        out_mem_ref[...] = jnp.where(sel, enc_ref[...], mem_ref[...])


def _tc_pass(pos, x, mean2, std2, W_enc, b2, memory):
    def big_map(i, p):
        return (jnp.where(i < NBLK, i, p[0] // BLK), 0)

    def const_map(i, p):
        return (0, 0)

    grid_spec = pltpu.PrefetchScalarGridSpec(
        num_scalar_prefetch=1,
        grid=(NBLK + 1,),
        in_specs=[
            pl.BlockSpec((1, IN_DIM), const_map),
            pl.BlockSpec((1, IN_DIM), const_map),
            pl.BlockSpec((1, IN_DIM), const_map),
            pl.BlockSpec((IN_DIM, OUT_DIM), const_map),
            pl.BlockSpec((1, OUT_DIM), const_map),
            pl.BlockSpec((BLK, OUT_DIM), big_map),
        ],
        out_specs=[
            pl.BlockSpec((1, 1), const_map),
            pl.BlockSpec((BLK, OUT_DIM), big_map),
        ],
        scratch_shapes=[
            pltpu.VMEM((1, OUT_DIM), jnp.float32),
            pltpu.SMEM((1,), jnp.float32),
        ],
    )

    return pl.pallas_call(
        _tc_body,
        grid_spec=grid_spec,
        out_shape=[
            jax.ShapeDtypeStruct((1, 1), jnp.float32),
            jax.ShapeDtypeStruct((MEM_LEN, OUT_DIM), jnp.float32),
        ],
        compiler_params=pltpu.CompilerParams(
            dimension_semantics=("arbitrary",),
        ),
    )(pos, x, mean2, std2, W_enc, b2, memory)


def _sc_copy(mem_data):
    mesh = plsc.ScalarSubcoreMesh(axis_name="c", num_cores=SC_CORES)

    @pl.kernel(
        out_type=jax.ShapeDtypeStruct((MEM_LEN, IN_DIM), jnp.float32),
        mesh=mesh,
        scratch_types=[pltpu.SemaphoreType.DMA],
    )
    def sc_kernel(md_hbm, o_hbm, sem):
        core = jax.lax.axis_index("c")
        base = core * SC_ROWS
        copies = []
        for j in range(SC_CHUNKS):
            s = base + j * SC_CHUNK_ROWS
            copies.append(pltpu.async_copy(
                md_hbm.at[pl.ds(s, SC_CHUNK_ROWS), :],
                o_hbm.at[pl.ds(s, SC_CHUNK_ROWS), :],
                sem,
            ))
        for c in copies:
            c.wait()

    return sc_kernel(mem_data)


def _md_fixup_body(md_any, loss_ref, x_ref, pos_ref, out_any, row_vmem, sem):
    @pl.when(loss_ref[0] <= BETA)
    def _():
        row_vmem[...] = x_ref[...]
        pltpu.async_copy(
            row_vmem, out_any.at[pl.ds(pos_ref[0], 1), :], sem,
        ).wait()


def _md_fixup(md_copied, loss2d, x, pos):
    return pl.pallas_call(
        _md_fixup_body,
        grid=(),
        in_specs=[
            pl.BlockSpec(memory_space=pltpu.MemorySpace.HBM),
            pl.BlockSpec(memory_space=pltpu.SMEM),
            pl.BlockSpec(memory_space=pltpu.VMEM),
            pl.BlockSpec(memory_space=pltpu.SMEM),
        ],
        out_specs=pl.BlockSpec(memory_space=pltpu.MemorySpace.HBM),
        out_shape=jax.ShapeDtypeStruct((MEM_LEN, IN_DIM), jnp.float32),
        scratch_shapes=[
            pltpu.VMEM((1, IN_DIM), jnp.float32),
            pltpu.SemaphoreType.DMA,
        ],
        input_output_aliases={0: 0},
    )(md_copied, loss2d.reshape(1), x, pos)


def kernel(x, mean, std, W_enc, b_enc, memory, mem_data, count):
    pos = jnp.asarray(count % MEM_LEN, jnp.int32).reshape(1)
    mean2 = mean.reshape(1, IN_DIM)
    std2 = std.reshape(1, IN_DIM)
    b2 = b_enc.reshape(1, OUT_DIM)

    md_copied = _sc_copy(mem_data)
    loss2d, new_memory = _tc_pass(pos, x, mean2, std2, W_enc, b2, memory)
    new_mem_data = _md_fixup(md_copied, loss2d, x, pos)

    return loss2d.reshape(()), new_memory, new_mem_data


# speedup vs baseline: 13.1896x; 13.1896x over previous
import jax
import jax.numpy as jnp
from jax.experimental import pallas as pl
from jax.experimental.pallas import tpu as pltpu
from jax.experimental.pallas import tpu_sc as plsc

IN_DIM = 128
OUT_DIM = 256
MEM_LEN = 100000
BETA = 2000.0

BLK = 5000
NBLK = MEM_LEN // BLK

SC_CORES = 2
SC_SUBCORES = 16
SC_WORKERS = SC_CORES * SC_SUBCORES
SC_CHUNK_ROWS = 400
SC_NCH = MEM_LEN // SC_CHUNK_ROWS
SC_NCH_PER_W = -(-SC_NCH // SC_WORKERS)


def _tc_body(pos_ref, x_ref, mean_ref, std_ref, w_ref, b_ref, mem_ref,
             loss_ref, out_mem_ref, enc_ref, min_ref):
    i = pl.program_id(0)

    @pl.when(i == 0)
    def _encode():
        xv = x_ref[...]
        std = std_ref[...]
        new = jnp.where(std == 0.0, 0.0, (xv - mean_ref[...]) / std)
        logits = jnp.dot(new, w_ref[...],
                         preferred_element_type=jnp.float32) + b_ref[...]
        m = jnp.max(logits)
        lse = jnp.log(jnp.sum(jnp.exp(logits - m))) + m
        enc_ref[...] = logits - lse
        min_ref[0] = jnp.inf

    @pl.when(i < NBLK)
    def _stream():
        blk = mem_ref[...]
        out_mem_ref[...] = blk
        d = jnp.sum(jnp.abs(blk - enc_ref[...]), axis=1)
        min_ref[0] = jnp.minimum(min_ref[0], jnp.min(d))

    @pl.when(i == NBLK)
    def _fixup():
        loss = min_ref[0]
        loss_ref[...] = jnp.full((1, 1), loss, jnp.float32)
        do_update = loss <= BETA
        r = pos_ref[0] % BLK
        row_sel = jax.lax.broadcasted_iota(jnp.int32, (BLK, 1), 0) == r
        sel = jnp.logical_and(do_update, row_sel)
        out_mem_ref[...] = jnp.where(sel, enc_ref[...], mem_ref[...])


def _tc_pass(pos, x, mean2, std2, W_enc, b2, memory):
    def big_map(i, p):
        return (jnp.where(i < NBLK, i, p[0] // BLK), 0)

    def const_map(i, p):
        return (0, 0)

    grid_spec = pltpu.PrefetchScalarGridSpec(
        num_scalar_prefetch=1,
        grid=(NBLK + 1,),
        in_specs=[
            pl.BlockSpec((1, IN_DIM), const_map),
            pl.BlockSpec((1, IN_DIM), const_map),
            pl.BlockSpec((1, IN_DIM), const_map),
            pl.BlockSpec((IN_DIM, OUT_DIM), const_map),
            pl.BlockSpec((1, OUT_DIM), const_map),
            pl.BlockSpec((BLK, OUT_DIM), big_map),
        ],
        out_specs=[
            pl.BlockSpec((1, 1), const_map),
            pl.BlockSpec((BLK, OUT_DIM), big_map),
        ],
        scratch_shapes=[
            pltpu.VMEM((1, OUT_DIM), jnp.float32),
            pltpu.SMEM((1,), jnp.float32),
        ],
    )

    return pl.pallas_call(
        _tc_body,
        grid_spec=grid_spec,
        out_shape=[
            jax.ShapeDtypeStruct((1, 1), jnp.float32),
            jax.ShapeDtypeStruct((MEM_LEN, OUT_DIM), jnp.float32),
        ],
        compiler_params=pltpu.CompilerParams(
            dimension_semantics=("arbitrary",),
        ),
    )(pos, x, mean2, std2, W_enc, b2, memory)


def _sc_copy(mem_data):
    mesh = plsc.VectorSubcoreMesh(core_axis_name="c", subcore_axis_name="s")

    @pl.kernel(
        out_type=jax.ShapeDtypeStruct((MEM_LEN, IN_DIM), jnp.float32),
        mesh=mesh,
        scratch_types=[pltpu.VMEM((SC_CHUNK_ROWS, IN_DIM), jnp.float32)],
    )
    def sc_kernel(md_hbm, o_hbm, buf):
        core = jax.lax.axis_index("c")
        sub = jax.lax.axis_index("s")
        w = core * SC_SUBCORES + sub

        @pl.loop(0, SC_NCH_PER_W)
        def _(k):
            c = w + k * SC_WORKERS

            @pl.when(c < SC_NCH)
            def _():
                s = c * SC_CHUNK_ROWS
                pltpu.sync_copy(md_hbm.at[pl.ds(s, SC_CHUNK_ROWS), :], buf)
                pltpu.sync_copy(buf, o_hbm.at[pl.ds(s, SC_CHUNK_ROWS), :])

    return sc_kernel(mem_data)


def _md_fixup_body(md_any, loss_ref, x_ref, pos_ref, out_any, row_vmem, sem):
    @pl.when(loss_ref[0] <= BETA)
    def _():
        row_vmem[...] = x_ref[...]
        pltpu.async_copy(
            row_vmem, out_any.at[pl.ds(pos_ref[0], 1), :], sem,
        ).wait()


def _md_fixup(md_copied, loss2d, x, pos):
    return pl.pallas_call(
        _md_fixup_body,
        grid=(),
        in_specs=[
            pl.BlockSpec(memory_space=pltpu.MemorySpace.HBM),
            pl.BlockSpec(memory_space=pltpu.SMEM),
            pl.BlockSpec(memory_space=pltpu.VMEM),
            pl.BlockSpec(memory_space=pltpu.SMEM),
        ],
        out_specs=pl.BlockSpec(memory_space=pltpu.MemorySpace.HBM),
        out_shape=jax.ShapeDtypeStruct((MEM_LEN, IN_DIM), jnp.float32),
        scratch_shapes=[
            pltpu.VMEM((1, IN_DIM), jnp.float32),
            pltpu.SemaphoreType.DMA,
        ],
        input_output_aliases={0: 0},
    )(md_copied, loss2d.reshape(1), x, pos)


def kernel(x, mean, std, W_enc, b_enc, memory, mem_data, count):
    pos = jnp.asarray(count % MEM_LEN, jnp.int32).reshape(1)
    mean2 = mean.reshape(1, IN_DIM)
    std2 = std.reshape(1, IN_DIM)
    b2 = b_enc.reshape(1, OUT_DIM)

    md_copied = _sc_copy(mem_data)
    loss2d, new_memory = _tc_pass(pos, x, mean2, std2, W_enc, b2, memory)
    new_mem_data = _md_fixup(md_copied, loss2d, x, pos)

    return loss2d.reshape(()), new_memory, new_mem_data
